# baseline (speedup 1.0000x reference)
import functools

import jax
import jax.numpy as jnp
from jax import lax
from jax.experimental import pallas as pl
from jax.experimental.pallas import tpu as pltpu
from jax.experimental.pallas import tpu_sc as plsc

DIM = 128
BATCH = 2
SEQ = 4096
CHUNK = 128
ROWS_PER_W = 256

_info = plsc.get_sparse_core_info()
_NC, _NS = _info.num_cores, _info.num_subcores
_NW = _NC * _NS
_W_PER_BATCH = SEQ // ROWS_PER_W

_mesh = plsc.VectorSubcoreMesh(core_axis_name="c", subcore_axis_name="s")


@functools.partial(
    pl.kernel,
    mesh=_mesh,
    out_type=(
        jax.ShapeDtypeStruct((BATCH, 1, SEQ, DIM), jnp.float32),
        jax.ShapeDtypeStruct((BATCH, 1, SEQ, DIM), jnp.float32),
    ),
    scratch_types=[
        pltpu.VMEM((ROWS_PER_W,), jnp.int32),
        pltpu.VMEM((ROWS_PER_W, DIM), jnp.float32),
        pltpu.VMEM((ROWS_PER_W, DIM), jnp.float32),
        pltpu.SemaphoreType.DMA,
        pltpu.SemaphoreType.DMA,
    ],
)
def _rope_gather(cos_hbm, sin_hbm, idx_hbm, cos_out, sin_out,
                 idx_v, cos_v, sin_v, gsem, ssem):
    wid = lax.axis_index("s") * _NC + lax.axis_index("c")
    b = wid // _W_PER_BATCH
    col = (wid % _W_PER_BATCH) * ROWS_PER_W
    pltpu.sync_copy(idx_hbm.at[b, pl.ds(col, ROWS_PER_W)], idx_v)
    gathers = []
    for j in range(ROWS_PER_W // CHUNK):
        sl = pl.ds(j * CHUNK, CHUNK)
        gathers.append(pltpu.async_copy(cos_hbm.at[idx_v.at[sl]], cos_v.at[sl], gsem))
        gathers.append(pltpu.async_copy(sin_hbm.at[idx_v.at[sl]], sin_v.at[sl], gsem))
    for c in gathers:
        c.wait()
    stores = [
        pltpu.async_copy(cos_v, cos_out.at[b, 0, pl.ds(col, ROWS_PER_W)], ssem),
        pltpu.async_copy(sin_v, sin_out.at[b, 0, pl.ds(col, ROWS_PER_W)], ssem),
    ]
    for c in stores:
        c.wait()


def kernel(x, position_ids, cos_cached, sin_cached):
    return _rope_gather(cos_cached, sin_cached, position_ids.astype(jnp.int32))
